# Initial kernel scaffold; baseline (speedup 1.0000x reference)
import jax
import jax.numpy as jnp
from jax.experimental import pallas as pl


def kernel(x, edge_index, W, att_src, att_dst, bias):
    raise NotImplementedError("write your pallas kernel here")


# speedup vs baseline: 18.2742x; 18.2742x over previous
import functools

import jax
import jax.numpy as jnp
from jax import lax
from jax.experimental import pallas as pl
from jax.experimental.pallas import tpu as pltpu
from jax.experimental.pallas import tpu_sc as plsc

N_NODES = 10000
N_PAD = 10240
D = 128
HEADS = 8
HD = 16
N_EDGES = 320000

NC = 2
NS = 16
NW = NC * NS
EDGES_PER_W = N_EDGES // NW
GROUPS_PER_W = EDGES_PER_W // 16
CHUNK_G = 125
N_CHUNKS = GROUPS_PER_W // CHUNK_G
STRIPE = N_PAD // NS

_HIGH = jax.lax.Precision.HIGHEST


def _pre_body(x_ref, w_ref, am_ref, ad_ref, h_ref, as_ref, adr_ref):
    h = jax.lax.dot(x_ref[...], w_ref[...], precision=_HIGH)
    h_ref[...] = h
    as_ref[...] = jax.lax.dot(h, am_ref[...], precision=_HIGH)
    adr_ref[...] = jax.lax.dot(h, ad_ref[...], precision=_HIGH)


def _dense_pre(x, W, AsM, AdM):
    blk = 1000
    grid = N_NODES // blk
    return pl.pallas_call(
        _pre_body,
        grid=(grid,),
        in_specs=[
            pl.BlockSpec((blk, D), lambda i: (i, 0)),
            pl.BlockSpec((D, D), lambda i: (0, 0)),
            pl.BlockSpec((D, 16), lambda i: (0, 0)),
            pl.BlockSpec((D, 16), lambda i: (0, 0)),
        ],
        out_specs=[
            pl.BlockSpec((blk, D), lambda i: (i, 0)),
            pl.BlockSpec((blk, 16), lambda i: (i, 0)),
            pl.BlockSpec((blk, 16), lambda i: (i, 0)),
        ],
        out_shape=[
            jax.ShapeDtypeStruct((N_NODES, D), jnp.float32),
            jax.ShapeDtypeStruct((N_NODES, 16), jnp.float32),
            jax.ShapeDtypeStruct((N_NODES, 16), jnp.float32),
        ],
    )(x, W, AsM, AdM)


def _edge_body(h_hbm, as_hbm, ad_hbm, src_hbm, dst_hbm,
               pm_hbm, pd_hbm,
               srcv, dstv, asv, adv, hv, msgv, sdenv,
               wbuf, dwbuf, accm, accd, sem):
    c = lax.axis_index("c")
    s = lax.axis_index("s")
    w = c * NS + s
    row_base = s * STRIPE

    zeros16 = jnp.zeros((16,), jnp.float32)
    for i in range(16):
        for k in range(D // 16):
            msgv[i, pl.ds(k * 16, 16)] = zeros16
        sdenv[i, :] = zeros16

    def zloop(i, carry):
        pltpu.sync_copy(msgv, accm.at[pl.ds(row_base + i * 16, 16)])
        pltpu.sync_copy(sdenv, accd.at[pl.ds(row_base + i * 16, 16)])
        return carry

    lax.fori_loop(0, STRIPE // 16, zloop, 0)
    plsc.subcore_barrier()

    iota16 = lax.iota(jnp.int32, 16)

    for chunk in range(N_CHUNKS):
        pltpu.sync_copy(src_hbm.at[w, chunk], srcv)
        pltpu.sync_copy(dst_hbm.at[w, chunk], dstv)

        def gbody(g, carry):
            si = srcv.at[g]
            di = dstv.at[g]
            cp_h = pltpu.async_copy(h_hbm.at[si], hv, sem)
            cp_a = pltpu.async_copy(as_hbm.at[si], asv, sem)
            cp_b = pltpu.async_copy(ad_hbm.at[di], adv, sem)
            cp_h.wait()
            cp_a.wait()
            cp_b.wait()
            for hh in range(HEADS):
                col = jnp.full((16,), hh, jnp.int32)
                a = plsc.load_gather(asv, [iota16, col])
                b = plsc.load_gather(adv, [iota16, col])
                e = a + b
                e = jnp.maximum(e, 0.2 * e)
                sv = jnp.exp(e)
                plsc.store_scatter(sdenv, [iota16, col], sv)
                for cc in range(HD):
                    colidx = jnp.full((16,), hh * HD + cc, jnp.int32)
                    hcol = plsc.load_gather(hv, [iota16, colidx])
                    plsc.store_scatter(msgv, [iota16, colidx], hcol * sv)
            pltpu.sync_copy(msgv, accm.at[di], add=True)
            pltpu.sync_copy(sdenv, accd.at[di], add=True)
            return carry

        lax.fori_loop(0, CHUNK_G, gbody, 0)

    plsc.subcore_barrier()

    def wloop(i, carry):
        rb = row_base + i * 64
        pltpu.sync_copy(accm.at[pl.ds(rb, 64)], wbuf)
        pltpu.sync_copy(wbuf, pm_hbm.at[c].at[pl.ds(rb, 64)])
        pltpu.sync_copy(accd.at[pl.ds(rb, 64)], dwbuf)
        pltpu.sync_copy(dwbuf, pd_hbm.at[c].at[pl.ds(rb, 64)])
        return carry

    lax.fori_loop(0, STRIPE // 64, wloop, 0)


def _edge_pass(h, as16, ad16, src2d, dst2d):
    mesh = plsc.VectorSubcoreMesh(core_axis_name="c", subcore_axis_name="s")
    fn = pl.kernel(
        _edge_body,
        out_type=[
            jax.ShapeDtypeStruct((NC, N_PAD, D), jnp.float32),
            jax.ShapeDtypeStruct((NC, N_PAD, 16), jnp.float32),
        ],
        mesh=mesh,
        scratch_types=[
            pltpu.VMEM((CHUNK_G, 16), jnp.int32),
            pltpu.VMEM((CHUNK_G, 16), jnp.int32),
            pltpu.VMEM((16, 16), jnp.float32),
            pltpu.VMEM((16, 16), jnp.float32),
            pltpu.VMEM((16, D), jnp.float32),
            pltpu.VMEM((16, D), jnp.float32),
            pltpu.VMEM((16, 16), jnp.float32),
            pltpu.VMEM((64, D), jnp.float32),
            pltpu.VMEM((64, 16), jnp.float32),
            pltpu.VMEM_SHARED((N_PAD, D), jnp.float32),
            pltpu.VMEM_SHARED((N_PAD, 16), jnp.float32),
            pltpu.SemaphoreType.DMA,
        ],
        compiler_params=pltpu.CompilerParams(
            needs_layout_passes=False, use_tc_tiling_on_sc=False),
    )
    return fn(h, as16, ad16, src2d, dst2d)


def _comb_body(pm_ref, pd_ref, h_ref, as_ref, ad_ref, b_ref, o_ref):
    e = as_ref[:, :HEADS] + ad_ref[:, :HEADS]
    e = jnp.maximum(e, 0.2 * e)
    sself = jnp.exp(e)
    den = pd_ref[0][:, :HEADS] + pd_ref[1][:, :HEADS] + sself
    rows = lax.broadcasted_iota(jnp.int32, (HEADS, D), 0)
    cols = lax.broadcasted_iota(jnp.int32, (HEADS, D), 1)
    expand = (cols // HD == rows).astype(jnp.float32)
    den128 = jax.lax.dot(den, expand, precision=_HIGH)
    s128 = jax.lax.dot(sself, expand, precision=_HIGH)
    msg = pm_ref[0] + pm_ref[1] + h_ref[...] * s128
    out = msg / den128 + b_ref[...]
    o_ref[...] = jnp.maximum(out, 0.0)


def _combine(pm, pd, h, as16, ad16, bias2d):
    blk = 1000
    grid = N_NODES // blk
    return pl.pallas_call(
        _comb_body,
        grid=(grid,),
        in_specs=[
            pl.BlockSpec((NC, blk, D), lambda i: (0, i, 0)),
            pl.BlockSpec((NC, blk, 16), lambda i: (0, i, 0)),
            pl.BlockSpec((blk, D), lambda i: (i, 0)),
            pl.BlockSpec((blk, 16), lambda i: (i, 0)),
            pl.BlockSpec((blk, 16), lambda i: (i, 0)),
            pl.BlockSpec((1, D), lambda i: (0, 0)),
        ],
        out_specs=pl.BlockSpec((blk, D), lambda i: (i, 0)),
        out_shape=jax.ShapeDtypeStruct((N_NODES, D), jnp.float32),
    )(pm, pd, h, as16, ad16, bias2d)


def kernel(x, edge_index, W, att_src, att_dst, bias):
    src2d = edge_index[0].astype(jnp.int32).reshape(NW, N_CHUNKS, CHUNK_G, 16)
    dst2d = edge_index[1].astype(jnp.int32).reshape(NW, N_CHUNKS, CHUNK_G, 16)

    eye = jnp.eye(HEADS, dtype=jnp.float32)
    a_s = att_src.reshape(HEADS, HD)
    a_d = att_dst.reshape(HEADS, HD)
    AsM = (a_s[:, :, None] * eye[:, None, :]).reshape(D, HEADS)
    AdM = (a_d[:, :, None] * eye[:, None, :]).reshape(D, HEADS)
    pad = jnp.zeros((D, 16 - HEADS), jnp.float32)
    AsM = jnp.concatenate([AsM, pad], axis=1)
    AdM = jnp.concatenate([AdM, pad], axis=1)

    h, as16, ad16 = _dense_pre(x, W, AsM, AdM)
    pm, pd = _edge_pass(h, as16, ad16, src2d, dst2d)
    bias2d = bias.reshape(1, D)
    return _combine(pm, pd, h, as16, ad16, bias2d)


# speedup vs baseline: 24.0560x; 1.3164x over previous
import functools

import jax
import jax.numpy as jnp
from jax import lax
from jax.experimental import pallas as pl
from jax.experimental.pallas import tpu as pltpu
from jax.experimental.pallas import tpu_sc as plsc

N_NODES = 10000
N_PAD = 10240
D = 128
HEADS = 8
HD = 16
N_EDGES = 320000

NC = 2
NS = 16
NW = NC * NS
EDGES_PER_W = N_EDGES // NW
G_TOT = EDGES_PER_W // 16
STRIPE = N_PAD // NS
WCHUNK = 32

_HIGH = jax.lax.Precision.HIGHEST


def _pre_body(x_ref, w_ref, am_ref, ad_ref, h_ref, as_ref, adr_ref):
    h = jax.lax.dot(x_ref[...], w_ref[...], precision=_HIGH)
    h_ref[...] = h
    as_ref[...] = jax.lax.dot(h, am_ref[...], precision=_HIGH)
    adr_ref[...] = jax.lax.dot(h, ad_ref[...], precision=_HIGH)


def _dense_pre(x, W, AsM, AdM):
    blk = 1000
    grid = N_NODES // blk
    return pl.pallas_call(
        _pre_body,
        grid=(grid,),
        in_specs=[
            pl.BlockSpec((blk, D), lambda i: (i, 0)),
            pl.BlockSpec((D, D), lambda i: (0, 0)),
            pl.BlockSpec((D, 16), lambda i: (0, 0)),
            pl.BlockSpec((D, 16), lambda i: (0, 0)),
        ],
        out_specs=[
            pl.BlockSpec((blk, D), lambda i: (i, 0)),
            pl.BlockSpec((blk, 16), lambda i: (i, 0)),
            pl.BlockSpec((blk, 16), lambda i: (i, 0)),
        ],
        out_shape=[
            jax.ShapeDtypeStruct((N_NODES, D), jnp.float32),
            jax.ShapeDtypeStruct((N_NODES, 16), jnp.float32),
            jax.ShapeDtypeStruct((N_NODES, 16), jnp.float32),
        ],
    )(x, W, AsM, AdM)


def _edge_body(h_hbm, as_hbm, ad_hbm, src_hbm, dst_hbm,
               pm_hbm, pd_hbm,
               srcv, dstv,
               hv0, hv1, asv0, asv1, adv0, adv1,
               msgv0, msgv1, sdenv0, sdenv1,
               wbuf, dwbuf, accm, accd,
               sg0, sg1, ss0, ss1):
    hv = (hv0, hv1)
    asv = (asv0, asv1)
    adv = (adv0, adv1)
    msgv = (msgv0, msgv1)
    sdenv = (sdenv0, sdenv1)
    sg = (sg0, sg1)
    ss = (ss0, ss1)

    c = lax.axis_index("c")
    s = lax.axis_index("s")
    w = c * NS + s
    row_base = s * STRIPE

    zeros16 = jnp.zeros((16,), jnp.float32)
    for i in range(16):
        for k in range(D // 16):
            msgv0[i, pl.ds(k * 16, 16)] = zeros16
        sdenv0[i, :] = zeros16
        sdenv1[i, :] = zeros16

    def zloop(i, carry):
        pltpu.sync_copy(msgv0, accm.at[pl.ds(row_base + i * 16, 16)])
        pltpu.sync_copy(sdenv0, accd.at[pl.ds(row_base + i * 16, 16)])
        return carry

    lax.fori_loop(0, STRIPE // 16, zloop, 0)
    plsc.subcore_barrier()

    pltpu.sync_copy(src_hbm.at[w], srcv)
    pltpu.sync_copy(dst_hbm.at[w], dstv)

    iota16 = lax.iota(jnp.int32, 16)

    def issue_gather(g, b):
        pltpu.async_copy(h_hbm.at[srcv.at[g]], hv[b], sg[b])
        pltpu.async_copy(as_hbm.at[srcv.at[g]], asv[b], sg[b])
        pltpu.async_copy(ad_hbm.at[dstv.at[g]], adv[b], sg[b])

    def wait_gather(g, b):
        pltpu.make_async_copy(h_hbm.at[srcv.at[g]], hv[b], sg[b]).wait()
        pltpu.make_async_copy(as_hbm.at[srcv.at[g]], asv[b], sg[b]).wait()
        pltpu.make_async_copy(ad_hbm.at[dstv.at[g]], adv[b], sg[b]).wait()

    def issue_scatter(g, b):
        pltpu.async_copy(msgv[b], accm.at[dstv.at[g]], ss[b], add=True)
        pltpu.async_copy(sdenv[b], accd.at[dstv.at[g]], ss[b], add=True)

    def wait_scatter(g, b):
        pltpu.make_async_copy(msgv[b], accm.at[dstv.at[g]], ss[b]).wait()
        pltpu.make_async_copy(sdenv[b], accd.at[dstv.at[g]], ss[b]).wait()

    def compute(b):
        for hh in range(HEADS):
            col = jnp.full((16,), hh, jnp.int32)
            a = plsc.load_gather(asv[b], [iota16, col])
            bb = plsc.load_gather(adv[b], [iota16, col])
            e = a + bb
            e = jnp.maximum(e, 0.2 * e)
            sv = jnp.exp(e)
            plsc.store_scatter(sdenv[b], [iota16, col], sv)
            for cc in range(HD):
                colidx = jnp.full((16,), hh * HD + cc, jnp.int32)
                hcol = plsc.load_gather(hv[b], [iota16, colidx])
                plsc.store_scatter(msgv[b], [iota16, colidx], hcol * sv)

    issue_gather(0, 0)
    issue_gather(1, 1)
    for g in (0, 1):
        b = g & 1
        wait_gather(g, b)
        compute(b)
        issue_scatter(g, b)
        issue_gather(g + 2, b)

    def pair_body(i, carry):
        for b in (0, 1):
            g = 2 * i + b
            wait_gather(g, b)
            wait_scatter(g - 2, b)
            compute(b)
            issue_scatter(g, b)

            @pl.when(g + 2 < G_TOT)
            def _():
                issue_gather(g + 2, b)
        return carry

    lax.fori_loop(1, (G_TOT - 1) // 2, pair_body, 0)

    wait_gather(G_TOT - 1, 0)
    wait_scatter(G_TOT - 3, 0)
    wait_scatter(G_TOT - 2, 1)
    compute(0)
    issue_scatter(G_TOT - 1, 0)
    wait_scatter(G_TOT - 1, 0)

    plsc.subcore_barrier()

    def wloop(i, carry):
        rb = row_base + i * WCHUNK
        pltpu.sync_copy(accm.at[pl.ds(rb, WCHUNK)], wbuf)
        pltpu.sync_copy(wbuf, pm_hbm.at[c].at[pl.ds(rb, WCHUNK)])
        pltpu.sync_copy(accd.at[pl.ds(rb, WCHUNK)], dwbuf)
        pltpu.sync_copy(dwbuf, pd_hbm.at[c].at[pl.ds(rb, WCHUNK)])
        return carry

    lax.fori_loop(0, STRIPE // WCHUNK, wloop, 0)


def _edge_pass(h, as16, ad16, src3d, dst3d):
    mesh = plsc.VectorSubcoreMesh(core_axis_name="c", subcore_axis_name="s")
    fn = pl.kernel(
        _edge_body,
        out_type=[
            jax.ShapeDtypeStruct((NC, N_PAD, D), jnp.float32),
            jax.ShapeDtypeStruct((NC, N_PAD, 16), jnp.float32),
        ],
        mesh=mesh,
        scratch_types=[
            pltpu.VMEM((G_TOT, 16), jnp.int32),
            pltpu.VMEM((G_TOT, 16), jnp.int32),
            pltpu.VMEM((16, D), jnp.float32),
            pltpu.VMEM((16, D), jnp.float32),
            pltpu.VMEM((16, 16), jnp.float32),
            pltpu.VMEM((16, 16), jnp.float32),
            pltpu.VMEM((16, 16), jnp.float32),
            pltpu.VMEM((16, 16), jnp.float32),
            pltpu.VMEM((16, D), jnp.float32),
            pltpu.VMEM((16, D), jnp.float32),
            pltpu.VMEM((16, 16), jnp.float32),
            pltpu.VMEM((16, 16), jnp.float32),
            pltpu.VMEM((WCHUNK, D), jnp.float32),
            pltpu.VMEM((WCHUNK, 16), jnp.float32),
            pltpu.VMEM_SHARED((N_PAD, D), jnp.float32),
            pltpu.VMEM_SHARED((N_PAD, 16), jnp.float32),
            pltpu.SemaphoreType.DMA,
            pltpu.SemaphoreType.DMA,
            pltpu.SemaphoreType.DMA,
            pltpu.SemaphoreType.DMA,
        ],
        compiler_params=pltpu.CompilerParams(
            needs_layout_passes=False, use_tc_tiling_on_sc=False),
    )
    return fn(h, as16, ad16, src3d, dst3d)


def _comb_body(pm_ref, pd_ref, h_ref, as_ref, ad_ref, b_ref, o_ref):
    e = as_ref[:, :HEADS] + ad_ref[:, :HEADS]
    e = jnp.maximum(e, 0.2 * e)
    sself = jnp.exp(e)
    den = pd_ref[0][:, :HEADS] + pd_ref[1][:, :HEADS] + sself
    rows = lax.broadcasted_iota(jnp.int32, (HEADS, D), 0)
    cols = lax.broadcasted_iota(jnp.int32, (HEADS, D), 1)
    expand = (cols // HD == rows).astype(jnp.float32)
    den128 = jax.lax.dot(den, expand, precision=_HIGH)
    s128 = jax.lax.dot(sself, expand, precision=_HIGH)
    msg = pm_ref[0] + pm_ref[1] + h_ref[...] * s128
    out = msg / den128 + b_ref[...]
    o_ref[...] = jnp.maximum(out, 0.0)


def _combine(pm, pd, h, as16, ad16, bias2d):
    blk = 1000
    grid = N_NODES // blk
    return pl.pallas_call(
        _comb_body,
        grid=(grid,),
        in_specs=[
            pl.BlockSpec((NC, blk, D), lambda i: (0, i, 0)),
            pl.BlockSpec((NC, blk, 16), lambda i: (0, i, 0)),
            pl.BlockSpec((blk, D), lambda i: (i, 0)),
            pl.BlockSpec((blk, 16), lambda i: (i, 0)),
            pl.BlockSpec((blk, 16), lambda i: (i, 0)),
            pl.BlockSpec((1, D), lambda i: (0, 0)),
        ],
        out_specs=pl.BlockSpec((blk, D), lambda i: (i, 0)),
        out_shape=jax.ShapeDtypeStruct((N_NODES, D), jnp.float32),
    )(pm, pd, h, as16, ad16, bias2d)


def kernel(x, edge_index, W, att_src, att_dst, bias):
    src3d = edge_index[0].astype(jnp.int32).reshape(NW, G_TOT, 16)
    dst3d = edge_index[1].astype(jnp.int32).reshape(NW, G_TOT, 16)

    eye = jnp.eye(HEADS, dtype=jnp.float32)
    a_s = att_src.reshape(HEADS, HD)
    a_d = att_dst.reshape(HEADS, HD)
    AsM = (a_s[:, :, None] * eye[:, None, :]).reshape(D, HEADS)
    AdM = (a_d[:, :, None] * eye[:, None, :]).reshape(D, HEADS)
    pad = jnp.zeros((D, 16 - HEADS), jnp.float32)
    AsM = jnp.concatenate([AsM, pad], axis=1)
    AdM = jnp.concatenate([AdM, pad], axis=1)

    h, as16, ad16 = _dense_pre(x, W, AsM, AdM)
    pm, pd = _edge_pass(h, as16, ad16, src3d, dst3d)
    bias2d = bias.reshape(1, D)
    return _combine(pm, pd, h, as16, ad16, bias2d)
